# baseline (speedup 1.0000x reference)
import functools

import jax
import jax.numpy as jnp
from jax.experimental import pallas as pl
from jax.experimental.pallas import tpu as pltpu

N_NODES = 512
D_FEAT = 256
N_HIDDEN = 256


def _dot(a, b):
    return jax.lax.dot_general(
        a, b, (((1,), (0,)), ((), ())),
        precision=jax.lax.Precision.DEFAULT,
        preferred_element_type=jnp.float32)


def _gcn_kernel(gids_ref, order_ref, x_ref, a_ref, mask_ref,
                Win_ref, bin_ref, W1_ref, b1_ref, W2_ref, b2_ref,
                W3_ref, b3_ref, out_ref, h_scratch):
    b = pl.program_id(0)
    prev = gids_ref[jnp.maximum(b - 1, 0)]
    is_new = jnp.logical_or(b == 0, gids_ref[b] != prev)

    @pl.when(is_new)
    def _compute():
        bf = jnp.bfloat16
        x = x_ref[0].astype(bf)
        a = a_ref[0].astype(bf)
        w_in = Win_ref[...].astype(bf)
        w1 = W1_ref[...].astype(bf)
        w2 = W2_ref[...].astype(bf)
        w3 = W3_ref[...].astype(bf)
        h0 = jax.nn.relu(_dot(x, w_in) + bin_ref[...])
        h = jax.nn.relu(_dot(_dot(a, h0.astype(bf)).astype(bf), w1) + b1_ref[...])
        h = jax.nn.relu(_dot(_dot(a, h.astype(bf)).astype(bf), w2) + b2_ref[...])
        h = jnp.tanh(_dot(_dot(a, h.astype(bf)).astype(bf), w3) + b3_ref[...])
        h_scratch[...] = h + h0

    m = mask_ref[0]
    denom = jnp.maximum(jnp.sum(m), 1.0)
    out_ref[0] = _dot(m, h_scratch[...]) / denom


def kernel(graph, coverpoint_mask, batch_xs, batch_as, W_in, b_in,
           W1, b1, W2, b2, W3, b3):
    B = graph.shape[0]
    order = jnp.argsort(graph).astype(jnp.int32)
    gids = jnp.take(graph, order).astype(jnp.int32)
    mask_f = coverpoint_mask.astype(jnp.float32).reshape(B, 1, N_NODES)

    grid_spec = pltpu.PrefetchScalarGridSpec(
        num_scalar_prefetch=2,
        grid=(B,),
        in_specs=[
            pl.BlockSpec((1, N_NODES, D_FEAT), lambda b, g, o: (g[b], 0, 0)),
            pl.BlockSpec((1, N_NODES, N_NODES), lambda b, g, o: (g[b], 0, 0)),
            pl.BlockSpec((1, 1, N_NODES), lambda b, g, o: (o[b], 0, 0)),
            pl.BlockSpec((D_FEAT, N_HIDDEN), lambda b, g, o: (0, 0)),
            pl.BlockSpec((1, N_HIDDEN), lambda b, g, o: (0, 0)),
            pl.BlockSpec((N_HIDDEN, N_HIDDEN), lambda b, g, o: (0, 0)),
            pl.BlockSpec((1, N_HIDDEN), lambda b, g, o: (0, 0)),
            pl.BlockSpec((N_HIDDEN, N_HIDDEN), lambda b, g, o: (0, 0)),
            pl.BlockSpec((1, N_HIDDEN), lambda b, g, o: (0, 0)),
            pl.BlockSpec((N_HIDDEN, N_HIDDEN), lambda b, g, o: (0, 0)),
            pl.BlockSpec((1, N_HIDDEN), lambda b, g, o: (0, 0)),
        ],
        out_specs=pl.BlockSpec((1, 1, N_HIDDEN), lambda b, g, o: (o[b], 0, 0)),
        scratch_shapes=[pltpu.VMEM((N_NODES, N_HIDDEN), jnp.float32)],
    )

    out = pl.pallas_call(
        _gcn_kernel,
        grid_spec=grid_spec,
        out_shape=jax.ShapeDtypeStruct((B, 1, N_HIDDEN), jnp.float32),
    )(gids, order, batch_xs, batch_as, mask_f,
      W_in, b_in.reshape(1, N_HIDDEN), W1, b1.reshape(1, N_HIDDEN),
      W2, b2.reshape(1, N_HIDDEN), W3, b3.reshape(1, N_HIDDEN))
    return out.reshape(B, N_HIDDEN)


# speedup vs baseline: 1.4255x; 1.4255x over previous
import jax
import jax.numpy as jnp
from jax.experimental import pallas as pl
from jax.experimental.pallas import tpu as pltpu

N_NODES = 512
D_FEAT = 256
N_HIDDEN = 256


def _dot(a, b):
    return jax.lax.dot_general(
        a, b, (((1,), (0,)), ((), ())),
        preferred_element_type=jnp.float32)


def _gcn_kernel(newf_ref, slot_ref, gidx_ref,
                xA_ref, aA_ref, xB_ref, aB_ref, maskA_ref, maskB_ref,
                Win_ref, bin_ref, W1_ref, b1_ref, W2_ref, b2_ref,
                W3_ref, b3_ref, out_ref, h_scratch):
    b = pl.program_id(0)
    qa = 2 * b
    qb = 2 * b + 1
    new_any = jnp.logical_or(newf_ref[qa] == 1, newf_ref[qb] == 1)
    sA = slot_ref[qa]
    sB = slot_ref[qb]

    @pl.when(new_any)
    def _compute():
        bf = jnp.bfloat16
        x2 = jnp.concatenate([xA_ref[0], xB_ref[0]], axis=0).astype(bf)
        aA = aA_ref[0].astype(bf)
        aB = aB_ref[0].astype(bf)
        h0 = jax.nn.relu(_dot(x2, Win_ref[...].astype(bf)) + bin_ref[...])
        h = h0
        for w_ref, b_ref, act in ((W1_ref, b1_ref, jax.nn.relu),
                                  (W2_ref, b2_ref, jax.nn.relu),
                                  (W3_ref, b3_ref, jnp.tanh)):
            hb = h.astype(bf)
            tA = _dot(aA, hb[:N_NODES])
            tB = _dot(aB, hb[N_NODES:])
            t = jnp.concatenate([tA, tB], axis=0).astype(bf)
            h = act(_dot(t, w_ref[...].astype(bf)) + b_ref[...])
        hf = h + h0
        h_scratch[sA] = hf[:N_NODES]
        h_scratch[sB] = hf[N_NODES:]

    mA = maskA_ref[0]
    mB = maskB_ref[0]
    outA = _dot(mA, h_scratch[sA]) / jnp.maximum(jnp.sum(mA), 1.0)
    outB = _dot(mB, h_scratch[sB]) / jnp.maximum(jnp.sum(mB), 1.0)
    out_ref[0] = outA
    out_ref[1] = outB


def kernel(graph, coverpoint_mask, batch_xs, batch_as, W_in, b_in,
           W1, b1, W2, b2, W3, b3):
    B = graph.shape[0]
    g = graph.astype(jnp.int32)
    eq = g[:, None] == g[None, :]
    firstocc = jnp.argmax(eq, axis=1).astype(jnp.int32)
    newf = (firstocc == jnp.arange(B, dtype=jnp.int32)).astype(jnp.int32)
    slot = (jnp.cumsum(newf) - 1)[firstocc].astype(jnp.int32)
    mask_f = coverpoint_mask.astype(jnp.float32).reshape(B, 1, N_NODES)

    grid_spec = pltpu.PrefetchScalarGridSpec(
        num_scalar_prefetch=3,
        grid=(B // 2,),
        in_specs=[
            pl.BlockSpec((1, N_NODES, D_FEAT),
                         lambda b, nf, sl, gi: (gi[2 * b], 0, 0)),
            pl.BlockSpec((1, N_NODES, N_NODES),
                         lambda b, nf, sl, gi: (gi[2 * b], 0, 0)),
            pl.BlockSpec((1, N_NODES, D_FEAT),
                         lambda b, nf, sl, gi: (gi[2 * b + 1], 0, 0)),
            pl.BlockSpec((1, N_NODES, N_NODES),
                         lambda b, nf, sl, gi: (gi[2 * b + 1], 0, 0)),
            pl.BlockSpec((1, 1, N_NODES),
                         lambda b, nf, sl, gi: (2 * b, 0, 0)),
            pl.BlockSpec((1, 1, N_NODES),
                         lambda b, nf, sl, gi: (2 * b + 1, 0, 0)),
            pl.BlockSpec((D_FEAT, N_HIDDEN), lambda b, nf, sl, gi: (0, 0)),
            pl.BlockSpec((1, N_HIDDEN), lambda b, nf, sl, gi: (0, 0)),
            pl.BlockSpec((N_HIDDEN, N_HIDDEN), lambda b, nf, sl, gi: (0, 0)),
            pl.BlockSpec((1, N_HIDDEN), lambda b, nf, sl, gi: (0, 0)),
            pl.BlockSpec((N_HIDDEN, N_HIDDEN), lambda b, nf, sl, gi: (0, 0)),
            pl.BlockSpec((1, N_HIDDEN), lambda b, nf, sl, gi: (0, 0)),
            pl.BlockSpec((N_HIDDEN, N_HIDDEN), lambda b, nf, sl, gi: (0, 0)),
            pl.BlockSpec((1, N_HIDDEN), lambda b, nf, sl, gi: (0, 0)),
        ],
        out_specs=pl.BlockSpec((2, 1, N_HIDDEN),
                               lambda b, nf, sl, gi: (b, 0, 0)),
        scratch_shapes=[pltpu.VMEM((B, N_NODES, N_HIDDEN), jnp.float32)],
    )

    out = pl.pallas_call(
        _gcn_kernel,
        grid_spec=grid_spec,
        out_shape=jax.ShapeDtypeStruct((B, 1, N_HIDDEN), jnp.float32),
    )(newf, slot, g, batch_xs, batch_as, batch_xs, batch_as, mask_f, mask_f,
      W_in, b_in.reshape(1, N_HIDDEN), W1, b1.reshape(1, N_HIDDEN),
      W2, b2.reshape(1, N_HIDDEN), W3, b3.reshape(1, N_HIDDEN))
    return out.reshape(B, N_HIDDEN)


# speedup vs baseline: 1.6239x; 1.1392x over previous
import jax
import jax.numpy as jnp
from jax.experimental import pallas as pl
from jax.experimental.pallas import tpu as pltpu

N_NODES = 512
D_FEAT = 256
N_HIDDEN = 256
_GROUP = 4


def _dot(a, b):
    return jax.lax.dot_general(
        a, b, (((1,), (0,)), ((), ())),
        preferred_element_type=jnp.float32)


def _gcn_kernel(newf_ref, slot_ref, gidx_ref, *refs):
    G = _GROUP
    x_refs = refs[0:2 * G:2]
    a_refs = refs[1:2 * G:2]
    mask_refs = refs[2 * G:3 * G]
    (Win_ref, bin_ref, W1_ref, b1_ref, W2_ref, b2_ref,
     W3_ref, b3_ref) = refs[3 * G:3 * G + 8]
    out_ref = refs[3 * G + 8]
    h_scratch = refs[3 * G + 9]

    b = pl.program_id(0)
    news = [newf_ref[G * b + j] == 1 for j in range(G)]
    slots = [slot_ref[G * b + j] for j in range(G)]
    new_any = news[0]
    for j in range(1, G):
        new_any = jnp.logical_or(new_any, news[j])

    @pl.when(new_any)
    def _compute():
        bf = jnp.bfloat16
        x2 = jnp.concatenate([r[0] for r in x_refs], axis=0).astype(bf)
        a_bf = [r[0].astype(bf) for r in a_refs]
        h0 = jax.nn.relu(_dot(x2, Win_ref[...].astype(bf)) + bin_ref[...])
        h = h0
        for w_ref, b_ref, act in ((W1_ref, b1_ref, jax.nn.relu),
                                  (W2_ref, b2_ref, jax.nn.relu),
                                  (W3_ref, b3_ref, jnp.tanh)):
            hb = h.astype(bf)
            ts = [_dot(a_bf[j], hb[j * N_NODES:(j + 1) * N_NODES])
                  for j in range(G)]
            t = jnp.concatenate(ts, axis=0).astype(bf)
            h = act(_dot(t, w_ref[...].astype(bf)) + b_ref[...])
        hf = h + h0
        for j in range(G):
            h_scratch[slots[j]] = hf[j * N_NODES:(j + 1) * N_NODES]

    for j in range(G):
        m = mask_refs[j][0]
        out_ref[j] = _dot(m, h_scratch[slots[j]]) / jnp.maximum(
            jnp.sum(m), 1.0)


def kernel(graph, coverpoint_mask, batch_xs, batch_as, W_in, b_in,
           W1, b1, W2, b2, W3, b3):
    B = graph.shape[0]
    G = _GROUP
    g = graph.astype(jnp.int32)
    eq = g[:, None] == g[None, :]
    slot = jnp.argmax(eq, axis=1).astype(jnp.int32)
    newf = (slot == jnp.arange(B, dtype=jnp.int32)).astype(jnp.int32)
    mask_f = coverpoint_mask.astype(jnp.float32).reshape(B, 1, N_NODES)

    xa_specs = []
    for j in range(G):
        xa_specs.append(pl.BlockSpec(
            (1, N_NODES, D_FEAT),
            lambda b, nf, sl, gi, j=j: (gi[G * b + j], 0, 0)))
        xa_specs.append(pl.BlockSpec(
            (1, N_NODES, N_NODES),
            lambda b, nf, sl, gi, j=j: (gi[G * b + j], 0, 0)))
    mask_specs = [
        pl.BlockSpec((1, 1, N_NODES),
                     lambda b, nf, sl, gi, j=j: (G * b + j, 0, 0))
        for j in range(G)
    ]
    w_specs = []
    for shape in ((D_FEAT, N_HIDDEN), (1, N_HIDDEN)) * 4:
        w_specs.append(pl.BlockSpec(shape, lambda b, nf, sl, gi: (0, 0)))

    grid_spec = pltpu.PrefetchScalarGridSpec(
        num_scalar_prefetch=3,
        grid=(B // G,),
        in_specs=xa_specs + mask_specs + w_specs,
        out_specs=pl.BlockSpec((G, 1, N_HIDDEN),
                               lambda b, nf, sl, gi: (b, 0, 0)),
        scratch_shapes=[pltpu.VMEM((B, N_NODES, N_HIDDEN), jnp.float32)],
    )

    xa_args = []
    for j in range(G):
        xa_args += [batch_xs, batch_as]

    out = pl.pallas_call(
        _gcn_kernel,
        grid_spec=grid_spec,
        out_shape=jax.ShapeDtypeStruct((B, 1, N_HIDDEN), jnp.float32),
    )(newf, slot, g, *xa_args, *([mask_f] * G),
      W_in, b_in.reshape(1, N_HIDDEN), W1, b1.reshape(1, N_HIDDEN),
      W2, b2.reshape(1, N_HIDDEN), W3, b3.reshape(1, N_HIDDEN))
    return out.reshape(B, N_HIDDEN)
